# baseline (speedup 1.0000x reference)
import jax
import jax.numpy as jnp
from jax import lax
from jax.experimental import pallas as pl
from jax.experimental.pallas import tpu as pltpu
from jax.experimental.pallas import tpu_sc as plsc

NC, NS = 2, 16
NW = NC * NS
B = 16384
NCHUNKS = 4
CB = B // NCHUNKS
RPW = CB // NW
SUB = 32
NSUB = RPW // SUB
EMBED = 128
BERT = 768
HALF = BERT // 2


def _pack_buf(src_f32, dst_packed):
    for k in range(BERT // 32):
        @plsc.parallel_loop(0, SUB, 1, unroll=4)
        def _(r, _k=k):
            a = src_f32[r, pl.ds(_k * 32, 16)]
            b = src_f32[r, pl.ds(_k * 32 + 16, 16)]
            ai = lax.bitcast_convert_type(a, jnp.int32)
            bi = lax.bitcast_convert_type(b, jnp.int32)
            ar = lax.shift_right_logical(ai + jnp.int32(0x8000), jnp.int32(16))
            br = (bi + jnp.int32(0x8000)) & jnp.int32(-65536)
            dst_packed[r, pl.ds(_k * 16, 16)] = (
                lax.bitcast_convert_type(ar | br, jnp.float32))


def _gather_body(users_hbm, items_hbm, user_table, bert_table,
                 user_out, bert_out,
                 uidx, iidx, bb0, bb1, pb0, pb1, ubuf,
                 gs_u, ss_u, gs0, gs1, ss0, ss1):
    wid = lax.axis_index("s") * NC + lax.axis_index("c")
    base = wid * RPW
    pltpu.sync_copy(users_hbm.at[pl.ds(base, RPW)], uidx)
    pltpu.sync_copy(items_hbm.at[pl.ds(base, RPW)], iidx)
    ug = pltpu.async_copy(user_table.at[uidx], ubuf, gs_u)
    bbufs, pbufs, gsems, ssems = (bb0, bb1), (pb0, pb1), (gs0, gs1), (ss0, ss1)
    gets = [pltpu.async_copy(bert_table.at[iidx.at[pl.ds(0, SUB)]],
                             bbufs[0], gsems[0]), None]
    stores = [None, None]
    for j in range(NSUB):
        pb = j % 2
        gets[pb].wait()
        if j + 1 < NSUB:
            nb = (j + 1) % 2
            gets[nb] = pltpu.async_copy(
                bert_table.at[iidx.at[pl.ds((j + 1) * SUB, SUB)]],
                bbufs[nb], gsems[nb])
        if stores[pb] is not None:
            stores[pb].wait()
            stores[pb] = None
        _pack_buf(bbufs[pb], pbufs[pb])
        stores[pb] = pltpu.async_copy(
            pbufs[pb], bert_out.at[pl.ds(base + j * SUB, SUB)], ssems[pb])
    ug.wait()
    us = pltpu.async_copy(ubuf, user_out.at[pl.ds(base, RPW)], ss_u)
    for st in stores:
        if st is not None:
            st.wait()
    us.wait()


_gather = pl.kernel(
    _gather_body,
    out_type=(jax.ShapeDtypeStruct((CB, EMBED), jnp.float32),
              jax.ShapeDtypeStruct((CB, HALF), jnp.float32)),
    mesh=plsc.VectorSubcoreMesh(core_axis_name="c", subcore_axis_name="s",
                                num_cores=NC, num_subcores=NS),
    scratch_types=[
        pltpu.VMEM((RPW,), jnp.int32),
        pltpu.VMEM((RPW,), jnp.int32),
        pltpu.VMEM((SUB, BERT), jnp.float32),
        pltpu.VMEM((SUB, BERT), jnp.float32),
        pltpu.VMEM((SUB, HALF), jnp.float32),
        pltpu.VMEM((SUB, HALF), jnp.float32),
        pltpu.VMEM((RPW, EMBED), jnp.float32),
        pltpu.SemaphoreType.DMA,
        pltpu.SemaphoreType.DMA,
        pltpu.SemaphoreType.DMA,
        pltpu.SemaphoreType.DMA,
        pltpu.SemaphoreType.DMA,
        pltpu.SemaphoreType.DMA,
    ],
)

BM = 1024


def _tc_body(user_ref, pk_ref, wlo_ref, whi_ref, b_ref, out_ref):
    p = lax.bitcast_convert_type(pk_ref[...], jnp.int32)
    lo = lax.bitcast_convert_type(lax.shift_left(p, jnp.int32(16)),
                                  jnp.float32)
    hi = lax.bitcast_convert_type(p & jnp.int32(-65536), jnp.float32)
    dn = (((1,), (1,)), ((), ()))
    news = (lax.dot_general(lo, wlo_ref[...], dn,
                            preferred_element_type=jnp.float32)
            + lax.dot_general(hi, whi_ref[...], dn,
                              preferred_element_type=jnp.float32))
    news = jax.nn.sigmoid(news + b_ref[...])
    out_ref[...] = jax.nn.sigmoid(jnp.sum(user_ref[...] * news, axis=1))


_tc = pl.pallas_call(
    _tc_body,
    grid=(CB // BM,),
    in_specs=[
        pl.BlockSpec((BM, EMBED), lambda i: (i, 0)),
        pl.BlockSpec((BM, HALF), lambda i: (i, 0)),
        pl.BlockSpec((EMBED, HALF), lambda i: (0, 0)),
        pl.BlockSpec((EMBED, HALF), lambda i: (0, 0)),
        pl.BlockSpec((1, EMBED), lambda i: (0, 0)),
    ],
    out_specs=pl.BlockSpec((BM,), lambda i: (i,)),
    out_shape=jax.ShapeDtypeStruct((CB,), jnp.float32),
)


def kernel(users, items, user_table, bert_table, W, b):
    b2 = b.reshape(1, EMBED)
    Wr = W.reshape(EMBED, BERT // 32, 32)
    Wlo = Wr[:, :, 0:16].reshape(EMBED, HALF)
    Whi = Wr[:, :, 16:32].reshape(EMBED, HALF)
    outs = []
    for c in range(NCHUNKS):
        ue, be = _gather(users[c * CB:(c + 1) * CB],
                         items[c * CB:(c + 1) * CB],
                         user_table, bert_table)
        outs.append(_tc(ue, be, Wlo, Whi, b2))
    return jnp.concatenate(outs)


# speedup vs baseline: 1.3193x; 1.3193x over previous
import jax
import jax.numpy as jnp
from jax import lax
from jax.experimental import pallas as pl
from jax.experimental.pallas import tpu as pltpu
from jax.experimental.pallas import tpu_sc as plsc

NC, NS = 2, 16
NW = NC * NS
B = 16384
NCHUNKS = 2
CB = B // NCHUNKS
RPW = CB // NW
SUB = 32
NSUB = RPW // SUB
EMBED = 128
BERT = 768
HALF = BERT // 2


def _pack_buf(src_f32, dst_packed):
    @plsc.parallel_loop(0, SUB, 1, unroll=2)
    def _(r):
        for k in range(BERT // 32):
            a = src_f32[r, pl.ds(k * 32, 16)]
            b = src_f32[r, pl.ds(k * 32 + 16, 16)]
            ai = lax.bitcast_convert_type(a, jnp.int32)
            bi = lax.bitcast_convert_type(b, jnp.int32)
            ar = lax.shift_right_logical(ai + jnp.int32(0x8000), jnp.int32(16))
            br = (bi + jnp.int32(0x8000)) & jnp.int32(-65536)
            dst_packed[r, pl.ds(k * 16, 16)] = (
                lax.bitcast_convert_type(ar | br, jnp.float32))


def _gather_body(users_hbm, items_hbm, user_table, bert_table,
                 user_out, bert_out,
                 uidx, iidx, bb0, bb1, pb0, pb1, ubuf,
                 gs_u, ss_u, gs0, gs1, ss0, ss1):
    wid = lax.axis_index("s") * NC + lax.axis_index("c")
    base = wid * RPW
    pltpu.sync_copy(users_hbm.at[pl.ds(base, RPW)], uidx)
    pltpu.sync_copy(items_hbm.at[pl.ds(base, RPW)], iidx)
    ug = pltpu.async_copy(user_table.at[uidx], ubuf, gs_u)
    bbufs, pbufs, gsems, ssems = (bb0, bb1), (pb0, pb1), (gs0, gs1), (ss0, ss1)
    gets = [pltpu.async_copy(bert_table.at[iidx.at[pl.ds(0, SUB)]],
                             bbufs[0], gsems[0]), None]
    stores = [None, None]
    for j in range(NSUB):
        pb = j % 2
        gets[pb].wait()
        if j + 1 < NSUB:
            nb = (j + 1) % 2
            gets[nb] = pltpu.async_copy(
                bert_table.at[iidx.at[pl.ds((j + 1) * SUB, SUB)]],
                bbufs[nb], gsems[nb])
        if stores[pb] is not None:
            stores[pb].wait()
            stores[pb] = None
        _pack_buf(bbufs[pb], pbufs[pb])
        stores[pb] = pltpu.async_copy(
            pbufs[pb], bert_out.at[pl.ds(base + j * SUB, SUB)], ssems[pb])
    ug.wait()
    us = pltpu.async_copy(ubuf, user_out.at[pl.ds(base, RPW)], ss_u)
    for st in stores:
        if st is not None:
            st.wait()
    us.wait()


_gather = pl.kernel(
    _gather_body,
    out_type=(jax.ShapeDtypeStruct((CB, EMBED), jnp.float32),
              jax.ShapeDtypeStruct((CB, HALF), jnp.float32)),
    mesh=plsc.VectorSubcoreMesh(core_axis_name="c", subcore_axis_name="s",
                                num_cores=NC, num_subcores=NS),
    scratch_types=[
        pltpu.VMEM((RPW,), jnp.int32),
        pltpu.VMEM((RPW,), jnp.int32),
        pltpu.VMEM((SUB, BERT), jnp.float32),
        pltpu.VMEM((SUB, BERT), jnp.float32),
        pltpu.VMEM((SUB, HALF), jnp.float32),
        pltpu.VMEM((SUB, HALF), jnp.float32),
        pltpu.VMEM((RPW, EMBED), jnp.float32),
        pltpu.SemaphoreType.DMA,
        pltpu.SemaphoreType.DMA,
        pltpu.SemaphoreType.DMA,
        pltpu.SemaphoreType.DMA,
        pltpu.SemaphoreType.DMA,
        pltpu.SemaphoreType.DMA,
    ],
)

BM = 1024


def _tc_body(user_ref, pk_ref, wlo_ref, whi_ref, b_ref, out_ref):
    p = lax.bitcast_convert_type(pk_ref[...], jnp.int32)
    lo = lax.bitcast_convert_type(lax.shift_left(p, jnp.int32(16)),
                                  jnp.float32)
    hi = lax.bitcast_convert_type(p & jnp.int32(-65536), jnp.float32)
    dn = (((1,), (1,)), ((), ()))
    news = (lax.dot_general(lo, wlo_ref[...], dn,
                            preferred_element_type=jnp.float32)
            + lax.dot_general(hi, whi_ref[...], dn,
                              preferred_element_type=jnp.float32))
    news = jax.nn.sigmoid(news + b_ref[...])
    out_ref[...] = jax.nn.sigmoid(jnp.sum(user_ref[...] * news, axis=1))


_tc = pl.pallas_call(
    _tc_body,
    grid=(CB // BM,),
    in_specs=[
        pl.BlockSpec((BM, EMBED), lambda i: (i, 0)),
        pl.BlockSpec((BM, HALF), lambda i: (i, 0)),
        pl.BlockSpec((EMBED, HALF), lambda i: (0, 0)),
        pl.BlockSpec((EMBED, HALF), lambda i: (0, 0)),
        pl.BlockSpec((1, EMBED), lambda i: (0, 0)),
    ],
    out_specs=pl.BlockSpec((BM,), lambda i: (i,)),
    out_shape=jax.ShapeDtypeStruct((CB,), jnp.float32),
)


def kernel(users, items, user_table, bert_table, W, b):
    b2 = b.reshape(1, EMBED)
    Wr = W.reshape(EMBED, BERT // 32, 32)
    Wlo = Wr[:, :, 0:16].reshape(EMBED, HALF)
    Whi = Wr[:, :, 16:32].reshape(EMBED, HALF)
    outs = []
    for c in range(NCHUNKS):
        ue, be = _gather(users[c * CB:(c + 1) * CB],
                         items[c * CB:(c + 1) * CB],
                         user_table, bert_table)
        outs.append(_tc(ue, be, Wlo, Whi, b2))
    return jnp.concatenate(outs)


# speedup vs baseline: 1.3264x; 1.0054x over previous
import jax
import jax.numpy as jnp
from jax import lax
from jax.experimental import pallas as pl
from jax.experimental.pallas import tpu as pltpu
from jax.experimental.pallas import tpu_sc as plsc

NC, NS = 2, 16
NW = NC * NS
B = 16384
NCHUNKS = 2
CB = B // NCHUNKS
RPW = CB // NW
SUB = 32
NSUB = RPW // SUB
EMBED = 128
BERT = 768
HALF = BERT // 2


def _pack_buf(src_f32, dst_packed):
    @plsc.parallel_loop(0, SUB, 1, unroll=2)
    def _(r):
        for k in range(BERT // 32):
            a = src_f32[r, pl.ds(k * 32, 16)]
            b = src_f32[r, pl.ds(k * 32 + 16, 16)]
            ai = lax.bitcast_convert_type(a, jnp.int32)
            bi = lax.bitcast_convert_type(b, jnp.int32)
            ar = lax.shift_right_logical(ai + jnp.int32(0x8000), jnp.int32(16))
            br = (bi + jnp.int32(0x8000)) & jnp.int32(-65536)
            dst_packed[r, pl.ds(k * 16, 16)] = (
                lax.bitcast_convert_type(ar | br, jnp.float32))


def _gather_body(users_hbm, items_hbm, user_table, bert_table,
                 user_out, bert_out,
                 uidx, iidx, bb0, bb1, pb0, pb1, ubuf,
                 gs_u, ss_u, gs0, gs1, ss0, ss1):
    wid = lax.axis_index("s") * NC + lax.axis_index("c")
    base = wid * RPW
    pltpu.sync_copy(users_hbm.at[pl.ds(base, RPW)], uidx)
    pltpu.sync_copy(items_hbm.at[pl.ds(base, RPW)], iidx)
    ug = pltpu.async_copy(user_table.at[uidx], ubuf, gs_u)
    bbufs, pbufs, gsems, ssems = (bb0, bb1), (pb0, pb1), (gs0, gs1), (ss0, ss1)
    gets = [pltpu.async_copy(bert_table.at[iidx.at[pl.ds(0, SUB)]],
                             bbufs[0], gsems[0]), None]
    stores = [None, None]
    for j in range(NSUB):
        pb = j % 2
        gets[pb].wait()
        if j + 1 < NSUB:
            nb = (j + 1) % 2
            gets[nb] = pltpu.async_copy(
                bert_table.at[iidx.at[pl.ds((j + 1) * SUB, SUB)]],
                bbufs[nb], gsems[nb])
        if stores[pb] is not None:
            stores[pb].wait()
            stores[pb] = None
        _pack_buf(bbufs[pb], pbufs[pb])
        stores[pb] = pltpu.async_copy(
            pbufs[pb], bert_out.at[pl.ds(base + j * SUB, SUB)], ssems[pb])
    ug.wait()
    us = pltpu.async_copy(ubuf, user_out.at[pl.ds(base, RPW)], ss_u)
    for st in stores:
        if st is not None:
            st.wait()
    us.wait()


_gather = pl.kernel(
    _gather_body,
    out_type=(jax.ShapeDtypeStruct((CB, EMBED), jnp.float32),
              jax.ShapeDtypeStruct((CB, HALF), jnp.float32)),
    mesh=plsc.VectorSubcoreMesh(core_axis_name="c", subcore_axis_name="s",
                                num_cores=NC, num_subcores=NS),
    scratch_types=[
        pltpu.VMEM((RPW,), jnp.int32),
        pltpu.VMEM((RPW,), jnp.int32),
        pltpu.VMEM((SUB, BERT), jnp.float32),
        pltpu.VMEM((SUB, BERT), jnp.float32),
        pltpu.VMEM((SUB, HALF), jnp.float32),
        pltpu.VMEM((SUB, HALF), jnp.float32),
        pltpu.VMEM((RPW, EMBED), jnp.float32),
        pltpu.SemaphoreType.DMA,
        pltpu.SemaphoreType.DMA,
        pltpu.SemaphoreType.DMA,
        pltpu.SemaphoreType.DMA,
        pltpu.SemaphoreType.DMA,
        pltpu.SemaphoreType.DMA,
    ],
)

BM = 1024


def _tc_body(user_ref, pk_ref, wlo_ref, whi_ref, b_ref, out_ref):
    p = lax.bitcast_convert_type(pk_ref[...], jnp.int32)
    lo = lax.bitcast_convert_type(lax.shift_left(p, jnp.int32(16)),
                                  jnp.float32).astype(jnp.bfloat16)
    hi = lax.bitcast_convert_type(p & jnp.int32(-65536),
                                  jnp.float32).astype(jnp.bfloat16)
    dn = (((1,), (1,)), ((), ()))
    news = (lax.dot_general(lo, wlo_ref[...], dn,
                            preferred_element_type=jnp.float32)
            + lax.dot_general(hi, whi_ref[...], dn,
                              preferred_element_type=jnp.float32))
    news = jax.nn.sigmoid(news + b_ref[...])
    out_ref[...] = jax.nn.sigmoid(jnp.sum(user_ref[...] * news, axis=1))


_tc = pl.pallas_call(
    _tc_body,
    grid=(CB // BM,),
    in_specs=[
        pl.BlockSpec((BM, EMBED), lambda i: (i, 0)),
        pl.BlockSpec((BM, HALF), lambda i: (i, 0)),
        pl.BlockSpec((EMBED, HALF), lambda i: (0, 0)),
        pl.BlockSpec((EMBED, HALF), lambda i: (0, 0)),
        pl.BlockSpec((1, EMBED), lambda i: (0, 0)),
    ],
    out_specs=pl.BlockSpec((BM,), lambda i: (i,)),
    out_shape=jax.ShapeDtypeStruct((CB,), jnp.float32),
)


def kernel(users, items, user_table, bert_table, W, b):
    b2 = b.reshape(1, EMBED)
    Wr = W.reshape(EMBED, BERT // 32, 32)
    Wlo = Wr[:, :, 0:16].reshape(EMBED, HALF).astype(jnp.bfloat16)
    Whi = Wr[:, :, 16:32].reshape(EMBED, HALF).astype(jnp.bfloat16)
    outs = []
    for c in range(NCHUNKS):
        ue, be = _gather(users[c * CB:(c + 1) * CB],
                         items[c * CB:(c + 1) * CB],
                         user_table, bert_table)
        outs.append(_tc(ue, be, Wlo, Whi, b2))
    return jnp.concatenate(outs)


# speedup vs baseline: 1.3530x; 1.0201x over previous
import jax
import jax.numpy as jnp
from jax import lax
from jax.experimental import pallas as pl
from jax.experimental.pallas import tpu as pltpu
from jax.experimental.pallas import tpu_sc as plsc

NC, NS = 2, 16
NW = NC * NS
B = 16384
NCHUNKS = 2
CB = B // NCHUNKS
RPW = CB // NW
SUB = 32
NSUB = RPW // SUB
EMBED = 128
BERT = 768
HALF = BERT // 2


def _pack_buf(src_f32, dst_packed):
    @plsc.parallel_loop(0, SUB, 1, unroll=2)
    def _(r):
        for k in range(BERT // 32):
            a = src_f32[r, pl.ds(k * 32, 16)]
            b = src_f32[r, pl.ds(k * 32 + 16, 16)]
            ai = lax.bitcast_convert_type(a, jnp.int32)
            bi = lax.bitcast_convert_type(b, jnp.int32)
            ar = lax.shift_right_logical(ai + jnp.int32(0x8000), jnp.int32(16))
            br = (bi + jnp.int32(0x8000)) & jnp.int32(-65536)
            dst_packed[r, pl.ds(k * 16, 16)] = (
                lax.bitcast_convert_type(ar | br, jnp.float32))


def _gather_body(users_hbm, items_hbm, user_table, bert_table,
                 user_out, bert_out,
                 uidx, iidx, bb0, bb1, pb0, pb1, ubuf,
                 gs_u, ss_u, gs0, gs1, ss0, ss1):
    wid = lax.axis_index("s") * NC + lax.axis_index("c")
    base = wid * RPW
    pltpu.sync_copy(users_hbm.at[pl.ds(base, RPW)], uidx)
    pltpu.sync_copy(items_hbm.at[pl.ds(base, RPW)], iidx)
    ug = pltpu.async_copy(user_table.at[uidx], ubuf, gs_u)
    bbufs, pbufs, gsems, ssems = (bb0, bb1), (pb0, pb1), (gs0, gs1), (ss0, ss1)
    gets = [pltpu.async_copy(bert_table.at[iidx.at[pl.ds(0, SUB)]],
                             bbufs[0], gsems[0]), None]
    stores = [None, None]
    for j in range(NSUB):
        pb = j % 2
        gets[pb].wait()
        if j + 1 < NSUB:
            nb = (j + 1) % 2
            gets[nb] = pltpu.async_copy(
                bert_table.at[iidx.at[pl.ds((j + 1) * SUB, SUB)]],
                bbufs[nb], gsems[nb])
        if stores[pb] is not None:
            stores[pb].wait()
            stores[pb] = None
        _pack_buf(bbufs[pb], pbufs[pb])
        stores[pb] = pltpu.async_copy(
            pbufs[pb], bert_out.at[pl.ds(base + j * SUB, SUB)], ssems[pb])
    ug.wait()
    us = pltpu.async_copy(ubuf, user_out.at[pl.ds(base, RPW)], ss_u)
    for st in stores:
        if st is not None:
            st.wait()
    us.wait()


_gather = pl.kernel(
    _gather_body,
    out_type=(jax.ShapeDtypeStruct((CB, EMBED), jnp.float32),
              jax.ShapeDtypeStruct((CB, HALF), jnp.float32)),
    mesh=plsc.VectorSubcoreMesh(core_axis_name="c", subcore_axis_name="s",
                                num_cores=NC, num_subcores=NS),
    scratch_types=[
        pltpu.VMEM((RPW,), jnp.int32),
        pltpu.VMEM((RPW,), jnp.int32),
        pltpu.VMEM((SUB, BERT), jnp.float32),
        pltpu.VMEM((SUB, BERT), jnp.float32),
        pltpu.VMEM((SUB, HALF), jnp.float32),
        pltpu.VMEM((SUB, HALF), jnp.float32),
        pltpu.VMEM((RPW, EMBED), jnp.float32),
        pltpu.SemaphoreType.DMA,
        pltpu.SemaphoreType.DMA,
        pltpu.SemaphoreType.DMA,
        pltpu.SemaphoreType.DMA,
        pltpu.SemaphoreType.DMA,
        pltpu.SemaphoreType.DMA,
    ],
)

BM = 1024


def _tc_body(user_ref, pk_ref, wlo_ref, whi_ref, b_ref, ones_ref, out_ref):
    p = lax.bitcast_convert_type(pk_ref[...], jnp.int32)
    lo = lax.bitcast_convert_type(lax.shift_left(p, jnp.int32(16)),
                                  jnp.float32).astype(jnp.bfloat16)
    hi = lax.bitcast_convert_type(p & jnp.int32(-65536),
                                  jnp.float32).astype(jnp.bfloat16)
    dn = (((1,), (1,)), ((), ()))
    news = (lax.dot_general(lo, wlo_ref[...], dn,
                            preferred_element_type=jnp.float32)
            + lax.dot_general(hi, whi_ref[...], dn,
                              preferred_element_type=jnp.float32))
    news = jax.nn.sigmoid(news + b_ref[...])
    prod = user_ref[...] * news
    dot = lax.dot_general(ones_ref[...], prod, dn,
                          preferred_element_type=jnp.float32)
    out_ref[...] = jax.nn.sigmoid(dot)


_tc = pl.pallas_call(
    _tc_body,
    grid=(CB // BM,),
    in_specs=[
        pl.BlockSpec((BM, EMBED), lambda i: (i, 0)),
        pl.BlockSpec((BM, HALF), lambda i: (i, 0)),
        pl.BlockSpec((EMBED, HALF), lambda i: (0, 0)),
        pl.BlockSpec((EMBED, HALF), lambda i: (0, 0)),
        pl.BlockSpec((1, EMBED), lambda i: (0, 0)),
        pl.BlockSpec((1, EMBED), lambda i: (0, 0)),
    ],
    out_specs=pl.BlockSpec((1, BM), lambda i: (0, i)),
    out_shape=jax.ShapeDtypeStruct((1, CB), jnp.float32),
)


def kernel(users, items, user_table, bert_table, W, b):
    b2 = b.reshape(1, EMBED)
    ones = jnp.ones((1, EMBED), jnp.float32)
    Wr = W.reshape(EMBED, BERT // 32, 32)
    Wlo = Wr[:, :, 0:16].reshape(EMBED, HALF).astype(jnp.bfloat16)
    Whi = Wr[:, :, 16:32].reshape(EMBED, HALF).astype(jnp.bfloat16)
    outs = []
    for c in range(NCHUNKS):
        ue, be = _gather(users[c * CB:(c + 1) * CB],
                         items[c * CB:(c + 1) * CB],
                         user_table, bert_table)
        outs.append(_tc(ue, be, Wlo, Whi, b2, ones))
    return jnp.concatenate(outs, axis=1).reshape(B)
